# baseline (speedup 1.0000x reference)
import functools
import jax
import jax.numpy as jnp
from jax import lax
from jax.experimental import pallas as pl
from jax.experimental import layout as _layout
from jax.experimental.pallas import tpu as pltpu, tpu_sc as plsc

_NUM_CODEBOOKS = 8
_VOCAB = 100000
_D = 64
_N = 1024 * 50 * 8
_NC, _NS, _L = 2, 16, 16
_NW = _NC * _NS
_CHUNK = 128
_NCHUNK = _N // (_NW * _CHUNK)
_T, _BH = 50, 8


def _make_gather():
    mesh = plsc.VectorSubcoreMesh(core_axis_name="c", subcore_axis_name="s")

    @functools.partial(
        pl.kernel,
        mesh=mesh,
        out_type=jax.ShapeDtypeStruct((_T, 8, 8, _BH, 8, 128), jnp.float32),
        compiler_params=pltpu.CompilerParams(needs_layout_passes=False),
        scratch_types=[
            pltpu.VMEM((_NCHUNK, _CHUNK), jnp.int32),
            pltpu.VMEM((_CHUNK,), jnp.int32),
            pltpu.VMEM((_CHUNK,), jnp.int32),
            pltpu.VMEM((_CHUNK,), jnp.int32),
            pltpu.VMEM((_CHUNK,), jnp.int32),
            pltpu.VMEM((_CHUNK, 128), jnp.float32),
            pltpu.VMEM((_CHUNK, 128), jnp.float32),
            pltpu.VMEM((_CHUNK, 128), jnp.float32),
            pltpu.VMEM((_CHUNK, 128), jnp.float32),
            pltpu.VMEM((_D, 128), jnp.float32),
            pltpu.VMEM((_D, 128), jnp.float32),
            pltpu.SemaphoreType.DMA,
            pltpu.SemaphoreType.DMA,
        ],
    )
    def gather_kernel(tok_hbm, table_hbm, out_hbm,
                      tokv, idx0, idx1, idx2, idx3,
                      rows0, rows1, rows2, rows3, ob0, ob1,
                      sem_g, sem_s):
        wid = lax.axis_index("s") * _NC + lax.axis_index("c")
        base = wid * _NCHUNK

        idxs = (idx0, idx1, idx2, idx3)
        rows = (rows0, rows1, rows2, rows3)
        obs = (ob0, ob1)

        pltpu.sync_copy(tok_hbm.at[wid], tokv)

        lanes = lax.iota(jnp.int32, _L)

        def stage_a(j, s):
            g = base + j
            c = lax.rem(g, 8)
            coff = c * _VOCAB
            for q in range(_CHUNK // _L):
                tok = tokv[j, pl.ds(q * _L, _L)]
                idxs[s][pl.ds(q * _L, _L)] = tok + coff

        def gather_copy(s):
            return pltpu.make_async_copy(table_hbm.at[idxs[s]], rows[s], sem_g)

        def store_copy(j, o, fh):
            g = base + j
            t = g // (8 * _BH)
            bh = lax.rem(g // 8, _BH)
            c = lax.rem(g, 8)
            return pltpu.make_async_copy(
                obs[o].at[pl.ds(fh * 8, 8)], out_hbm.at[t, c, fh, bh], sem_s)

        def extract(s, o):
            bvs = [lanes + q * _L for q in range(8)]

            def frow(f, fv):
                for q in range(8):
                    v = plsc.load_gather(rows[s], [bvs[q], fv])
                    obs[o][f, pl.ds(q * _L, _L)] = v
                return fv + 1

            lax.fori_loop(0, _D, frow, lanes * 0)

        for s in range(4):
            stage_a(s, s)
            gather_copy(s).start()

        def body(k, _):
            for s in range(4):
                j = 4 * k + s
                o = s % 2

                @pl.when(j >= 2)
                def _():
                    for fh in range(8):
                        store_copy(j - 2, o, fh).wait()

                gather_copy(s).wait()
                extract(s, o)
                for fh in range(8):
                    store_copy(j, o, fh).start()

                @pl.when(k < (_NCHUNK // 4) - 1)
                def _():
                    stage_a(j + 4, s)
                    gather_copy(s).start()
            return 0

        lax.fori_loop(0, _NCHUNK // 4, body, 0)

        for j in (_NCHUNK - 2, _NCHUNK - 1):
            for fh in range(8):
                store_copy(j, j % 2, fh).wait()

    return gather_kernel


_gather = _make_gather()


@jax.jit
def kernel(in_tokens, embedding_weight):
    tok = (in_tokens.transpose(1, 0, 2)
           .reshape(_T, _BH, _CHUNK, 8)
           .transpose(0, 1, 3, 2)
           .reshape(_NW, _NCHUNK, _CHUNK))
    table2 = jnp.pad(embedding_weight, ((0, 0), (0, _D)))
    out6 = _gather(tok, table2)
    final = (out6.transpose(3, 5, 0, 1, 2, 4)
             .reshape(1024, _T, _NUM_CODEBOOKS, _D))
    return _layout.with_layout_constraint(
        final,
        _layout.Layout(major_to_minor=(1, 2, 3, 0), tiling=((8, 128),)))


# speedup vs baseline: 1.9487x; 1.9487x over previous
import functools
import jax
import jax.numpy as jnp
from jax import lax
from jax.experimental import pallas as pl
from jax.experimental import layout as _layout
from jax.experimental.pallas import tpu as pltpu, tpu_sc as plsc

_NUM_CODEBOOKS = 8
_VOCAB = 100000
_D = 64
_N = 1024 * 50 * 8
_NC, _NS, _L = 2, 16, 16
_NW = _NC * _NS
_CHUNK = 128
_NCHUNK = _N // (_NW * _CHUNK)
_T, _BH = 50, 8


def _make_gather():
    mesh = plsc.VectorSubcoreMesh(core_axis_name="c", subcore_axis_name="s")

    @functools.partial(
        pl.kernel,
        mesh=mesh,
        out_type=jax.ShapeDtypeStruct((_T, 8, 8, _BH, 8, 128), jnp.float32),
        compiler_params=pltpu.CompilerParams(needs_layout_passes=False),
        scratch_types=[
            pltpu.VMEM((_NCHUNK, _CHUNK), jnp.int32),
            pltpu.VMEM((_CHUNK,), jnp.int32),
            pltpu.VMEM((_CHUNK,), jnp.int32),
            pltpu.VMEM((_CHUNK,), jnp.int32),
            pltpu.VMEM((_CHUNK,), jnp.int32),
            pltpu.VMEM((_CHUNK, 128), jnp.float32),
            pltpu.VMEM((_CHUNK, 128), jnp.float32),
            pltpu.VMEM((_CHUNK, 128), jnp.float32),
            pltpu.VMEM((_CHUNK, 128), jnp.float32),
            pltpu.VMEM((_D, 128), jnp.float32),
            pltpu.VMEM((_D, 128), jnp.float32),
            pltpu.SemaphoreType.DMA,
            pltpu.SemaphoreType.DMA,
        ],
    )
    def gather_kernel(tok_hbm, table_hbm, out_hbm,
                      tokv, idx0, idx1, idx2, idx3,
                      rows0, rows1, rows2, rows3, ob0, ob1,
                      sem_g, sem_s):
        wid = lax.axis_index("s") * _NC + lax.axis_index("c")
        base = wid * _NCHUNK

        idxs = (idx0, idx1, idx2, idx3)
        rows = (rows0, rows1, rows2, rows3)
        obs = (ob0, ob1)

        pltpu.sync_copy(tok_hbm.at[wid], tokv)

        lanes = lax.iota(jnp.int32, _L)

        def stage_a(j, s):
            g = base + j
            c = lax.rem(g, 8)
            coff = c * _VOCAB
            for q in range(_CHUNK // _L):
                tok = tokv[j, pl.ds(q * _L, _L)]
                idxs[s][pl.ds(q * _L, _L)] = tok + coff

        def gather_copy(s):
            return pltpu.make_async_copy(table_hbm.at[idxs[s]], rows[s], sem_g)

        def store_copy(j, o, fh):
            g = base + j
            t = g // (8 * _BH)
            bh = lax.rem(g // 8, _BH)
            c = lax.rem(g, 8)
            return pltpu.make_async_copy(
                obs[o].at[pl.ds(fh * 8, 8)], out_hbm.at[t, c, fh, bh], sem_s)

        def extract(s, o):
            bvs = [lanes + q * _L for q in range(8)]

            @functools.partial(plsc.parallel_loop, 0, _D,
                               unroll=8, carry=lanes * 0)
            def _(f, fv):
                for q in range(8):
                    v = plsc.load_gather(rows[s], [bvs[q], fv])
                    obs[o][f, pl.ds(q * _L, _L)] = v
                return fv + 1

        for s in range(4):
            stage_a(s, s)
            gather_copy(s).start()

        def body(k, _):
            for s in range(4):
                j = 4 * k + s
                o = s % 2

                @pl.when(j >= 2)
                def _():
                    for fh in range(8):
                        store_copy(j - 2, o, fh).wait()

                gather_copy(s).wait()
                extract(s, o)
                for fh in range(8):
                    store_copy(j, o, fh).start()

                @pl.when(k < (_NCHUNK // 4) - 1)
                def _():
                    stage_a(j + 4, s)
                    gather_copy(s).start()
            return 0

        lax.fori_loop(0, _NCHUNK // 4, body, 0)

        for j in (_NCHUNK - 2, _NCHUNK - 1):
            for fh in range(8):
                store_copy(j, j % 2, fh).wait()

    return gather_kernel


_gather = _make_gather()


@jax.jit
def kernel(in_tokens, embedding_weight):
    tok = (in_tokens.transpose(1, 0, 2)
           .reshape(_T, _BH, _CHUNK, 8)
           .transpose(0, 1, 3, 2)
           .reshape(_NW, _NCHUNK, _CHUNK))
    table2 = jnp.pad(embedding_weight, ((0, 0), (0, _D)))
    out6 = _gather(tok, table2)
    final = (out6.transpose(3, 5, 0, 1, 2, 4)
             .reshape(1024, _T, _NUM_CODEBOOKS, _D))
    return _layout.with_layout_constraint(
        final,
        _layout.Layout(major_to_minor=(1, 2, 3, 0), tiling=((8, 128),)))


# speedup vs baseline: 1.9546x; 1.0031x over previous
import functools
import jax
import jax.numpy as jnp
from jax import lax
from jax.experimental import pallas as pl
from jax.experimental import layout as _layout
from jax.experimental.pallas import tpu as pltpu, tpu_sc as plsc

_NUM_CODEBOOKS = 8
_VOCAB = 100000
_D = 64
_N = 1024 * 50 * 8
_NC, _NS, _L = 2, 16, 16
_NW = _NC * _NS
_CHUNK = 128
_NCHUNK = _N // (_NW * _CHUNK)
_T, _BH = 50, 8


def _make_gather():
    mesh = plsc.VectorSubcoreMesh(core_axis_name="c", subcore_axis_name="s")

    @functools.partial(
        pl.kernel,
        mesh=mesh,
        out_type=jax.ShapeDtypeStruct((_T, 8, 8, _BH, 8, 128), jnp.float32),
        compiler_params=pltpu.CompilerParams(needs_layout_passes=False),
        scratch_types=[
            pltpu.VMEM((_NCHUNK, _CHUNK), jnp.int32),
            pltpu.VMEM((_CHUNK,), jnp.int32),
            pltpu.VMEM((_CHUNK,), jnp.int32),
            pltpu.VMEM((_CHUNK,), jnp.int32),
            pltpu.VMEM((_CHUNK,), jnp.int32),
            pltpu.VMEM((_CHUNK, 128), jnp.float32),
            pltpu.VMEM((_CHUNK, 128), jnp.float32),
            pltpu.VMEM((_CHUNK, 128), jnp.float32),
            pltpu.VMEM((_CHUNK, 128), jnp.float32),
            pltpu.VMEM((_D, 128), jnp.float32),
            pltpu.VMEM((_D, 128), jnp.float32),
            pltpu.SemaphoreType.DMA,
            pltpu.SemaphoreType.DMA,
        ],
    )
    def gather_kernel(tok_hbm, table_hbm, out_hbm,
                      tokv, idx0, idx1, idx2, idx3,
                      rows0, rows1, rows2, rows3, ob0, ob1,
                      sem_g, sem_s):
        wid = lax.axis_index("s") * _NC + lax.axis_index("c")
        base = wid * _NCHUNK

        idxs = (idx0, idx1, idx2, idx3)
        rows = (rows0, rows1, rows2, rows3)
        obs = (ob0, ob1)

        pltpu.sync_copy(tok_hbm.at[wid], tokv)

        lanes = lax.iota(jnp.int32, _L)

        def stage_a(j, s):
            g = base + j
            c = lax.rem(g, 8)
            coff = c * _VOCAB
            for q in range(_CHUNK // _L):
                tok = tokv[j, pl.ds(q * _L, _L)]
                idxs[s][pl.ds(q * _L, _L)] = tok + coff

        def gather_copy(s):
            return pltpu.make_async_copy(table_hbm.at[idxs[s]], rows[s], sem_g)

        def store_copy(j, o, fh):
            g = base + j
            t = g // (8 * _BH)
            bh = lax.rem(g // 8, _BH)
            c = lax.rem(g, 8)
            return pltpu.make_async_copy(
                obs[o].at[pl.ds(fh * 8, 8)], out_hbm.at[t, c, fh, bh], sem_s)

        def extract(s, o):
            bvs = [lanes + q * _L for q in range(8)]

            zero = lanes * 0

            @functools.partial(plsc.parallel_loop, 0, _D, unroll=8)
            def _(f):
                fv = zero + f
                for q in range(8):
                    v = plsc.load_gather(rows[s], [bvs[q], fv])
                    obs[o][f, pl.ds(q * _L, _L)] = v

        for s in range(4):
            stage_a(s, s)
            gather_copy(s).start()

        def body(k, _):
            for s in range(4):
                j = 4 * k + s
                o = s % 2

                @pl.when(j >= 2)
                def _():
                    for fh in range(8):
                        store_copy(j - 2, o, fh).wait()

                gather_copy(s).wait()
                extract(s, o)
                for fh in range(8):
                    store_copy(j, o, fh).start()

                @pl.when(k < (_NCHUNK // 4) - 1)
                def _():
                    stage_a(j + 4, s)
                    gather_copy(s).start()
            return 0

        lax.fori_loop(0, _NCHUNK // 4, body, 0)

        for j in (_NCHUNK - 2, _NCHUNK - 1):
            for fh in range(8):
                store_copy(j, j % 2, fh).wait()

    return gather_kernel


_gather = _make_gather()


@jax.jit
def kernel(in_tokens, embedding_weight):
    tok = (in_tokens.transpose(1, 0, 2)
           .reshape(_T, _BH, _CHUNK, 8)
           .transpose(0, 1, 3, 2)
           .reshape(_NW, _NCHUNK, _CHUNK))
    table2 = jnp.pad(embedding_weight, ((0, 0), (0, _D)))
    out6 = _gather(tok, table2)
    final = (out6.transpose(3, 5, 0, 1, 2, 4)
             .reshape(1024, _T, _NUM_CODEBOOKS, _D))
    return _layout.with_layout_constraint(
        final,
        _layout.Layout(major_to_minor=(1, 2, 3, 0), tiling=((8, 128),)))
